# baseline (speedup 1.0000x reference)
import dataclasses
import functools

import jax
import jax.numpy as jnp
from jax import lax
from jax.experimental import pallas as pl
from jax.experimental.pallas import tpu as pltpu
from jax.experimental.pallas import tpu_sc as plsc

N_COARSE = 64
N_FINE = 128
N_ALL = N_COARSE + N_FINE
MIN_DEPTH = 2.0
MAX_DEPTH = 6.0
NW = 32
CHUNK = 128
GROUPS = CHUNK // 16
PTS_BS = 1024


def _sc_sampler(densT, zT, deltaT, uT, zeT, zsT,
                dens_s, z_s, delta_s, u_s, ze_s, cdf_s, hist_s, outz_s):
    B = densT.shape[1]
    rays_per_w = B // NW
    nchunk = rays_per_w // CHUNK
    w_id = lax.axis_index("c") * 16 + lax.axis_index("s")
    lane = lax.iota(jnp.int32, 16)
    izeros = jnp.zeros((16,), jnp.int32)

    @pl.loop(0, N_COARSE)
    def _hz(j):
        hist_s[j, :] = izeros

    @pl.loop(0, nchunk)
    def _chunk(ci):
        col = w_id * rays_per_w + ci * CHUNK
        pltpu.sync_copy(densT.at[:, pl.ds(col, CHUNK)], dens_s)
        pltpu.sync_copy(zT.at[:, pl.ds(col, CHUNK)], z_s)
        pltpu.sync_copy(deltaT.at[:, pl.ds(col, CHUNK)], delta_s)
        pltpu.sync_copy(uT.at[:, pl.ds(col, CHUNK)], u_s)
        pltpu.sync_copy(zeT.at[:, pl.ds(col, CHUNK)],
                        ze_s.at[pl.ds(0, N_COARSE + 1)])

        for g in range(GROUPS):
            c0 = g * 16
            glane = lane + c0

            cdf_s[0, :] = jnp.zeros((16,), jnp.float32)

            def wstep(j, carry):
                T, wsum = carry
                x = dens_s[j, pl.ds(c0, 16)] * delta_s[j, pl.ds(c0, 16)]
                e = jnp.exp(-x)
                wj = (1.0 - e) * T + 1e-05
                wsum = wsum + wj
                cdf_s[j + 1, :] = wsum
                return T * (e + 1e-10), wsum

            ones = jnp.full((16,), 1.0, jnp.float32)
            _, wsum = lax.fori_loop(
                0, N_COARSE, wstep, (ones, jnp.zeros((16,), jnp.float32)))
            inv = ones / wsum
            thresh = 1e-05 * wsum

            h_grid = (MAX_DEPTH - MIN_DEPTH) / (N_COARSE - 1)
            s0 = MIN_DEPTH + 0.5 * h_grid
            inv_h = 1.0 / h_grid
            ione = jnp.full((16,), 1, jnp.int32)

            @pl.loop(0, N_FINE, step=4)
            def _q(q0):
                for q in (q0, q0 + 1, q0 + 2, q0 + 3):
                    uu = u_s[q, pl.ds(c0, 16)]
                    uw = uu * wsum
                    k = jnp.zeros((16,), jnp.int32)
                    for b in (64, 32, 16, 8, 4, 2, 1):
                        cand = jnp.minimum(k + b, N_COARSE)
                        val = plsc.load_gather(cdf_s, [cand, lane])
                        k = jnp.where(val <= uw, cand, k)
                    k1 = jnp.minimum(k + 1, N_COARSE)
                    cdf0 = plsc.load_gather(cdf_s, [k, lane])
                    cdf1 = plsc.load_gather(cdf_s, [k1, lane])
                    e0 = plsc.load_gather(ze_s, [k, glane])
                    e1 = plsc.load_gather(ze_s, [k1, glane])
                    dn = cdf1 - cdf0
                    num = uw - cdf0
                    t = jnp.where(dn < thresh, num * inv, num / dn)
                    fz = e0 + t * (e1 - e0)
                    m = ((fz - s0) * inv_h).astype(jnp.int32) + 1
                    m = jnp.clip(m, 1, N_COARSE - 2)
                    za = plsc.load_gather(z_s, [m - 1, glane])
                    zb = plsc.load_gather(z_s, [m, glane])
                    zc = plsc.load_gather(z_s, [m + 1, glane])
                    cnt = (m - 1) + (za <= fz).astype(jnp.int32) \
                        + (zb <= fz).astype(jnp.int32) \
                        + (zc <= fz).astype(jnp.int32)
                    plsc.addupdate_scatter(hist_s, [cnt, lane], ione)
                    plsc.store_scatter(outz_s, [cnt + q, glane], fz)

            def istep(i, H):
                H = H + hist_s[i, :]
                hist_s[i, :] = izeros
                zi = z_s[i, pl.ds(c0, 16)]
                plsc.store_scatter(outz_s, [H + i, glane], zi)
                return H

            lax.fori_loop(0, N_COARSE, istep, jnp.zeros((16,), jnp.int32))

        pltpu.sync_copy(outz_s, zsT.at[:, pl.ds(col, CHUNK)])


def _pts_body(zsT_ref, oT_ref, dT_ref, pts_ref):
    zs = zsT_ref[...]
    o = oT_ref[...]
    d = dT_ref[...]
    pts_ref[...] = o[:, None, :] + d[:, None, :] * zs[None, :, :]


def _constants(B):
    rkey = jax.random.key(12345)
    rk1, rk2 = jax.random.split(rkey)
    zlin = jnp.linspace(MIN_DEPTH, MAX_DEPTH, N_COARSE, dtype=jnp.float32)
    z_vals = jnp.broadcast_to(zlin, (B, N_COARSE))
    mids = 0.5 * (z_vals[..., 1:] + z_vals[..., :-1])
    upper = jnp.concatenate([mids, z_vals[..., -1:]], axis=-1)
    lower = jnp.concatenate([z_vals[..., :1], mids], axis=-1)
    t_rand = jax.random.uniform(rk1, z_vals.shape, dtype=jnp.float32)
    z_vals = lower + (upper - lower) * t_rand
    u = jax.random.uniform(rk2, (B, N_FINE), dtype=jnp.float32)
    u_asc = jnp.sort(u, axis=-1)
    delta = jnp.concatenate(
        [z_vals[..., 1:] - z_vals[..., :-1],
         jnp.full((B, 1), 1e10, jnp.float32)], axis=-1)
    ze_mid = 0.5 * (z_vals[..., 1:] + z_vals[..., :-1])
    z_edges = jnp.concatenate(
        [
            z_vals[..., :1] - 0.5 * (z_vals[..., 1:2] - z_vals[..., :1]),
            ze_mid,
            z_vals[..., -1:] + 0.5 * (z_vals[..., -1:] - z_vals[..., -2:-1]),
        ],
        axis=-1,
    )
    return (z_vals.T, delta.T, u_asc.T, z_edges.T)


def kernel(origins, directions, coarse_density):
    B = origins.shape[0]
    with jax.ensure_compile_time_eval():
        zT, deltaT, uT, zeT = _constants(B)

    densT = coarse_density[..., 0].T

    cp = pltpu.CompilerParams()
    if "needs_layout_passes" in pltpu.CompilerParams.__dataclass_fields__:
        cp = dataclasses.replace(cp, needs_layout_passes=False)
    sck = functools.partial(
        pl.kernel,
        out_type=jax.ShapeDtypeStruct((N_ALL, B), jnp.float32),
        compiler_params=cp,
        mesh=plsc.VectorSubcoreMesh(core_axis_name="c", subcore_axis_name="s"),
        scratch_types=[
            pltpu.VMEM((N_COARSE, CHUNK), jnp.float32),
            pltpu.VMEM((N_COARSE, CHUNK), jnp.float32),
            pltpu.VMEM((N_COARSE, CHUNK), jnp.float32),
            pltpu.VMEM((N_FINE, CHUNK), jnp.float32),
            pltpu.VMEM((N_COARSE + 8, CHUNK), jnp.float32),
            pltpu.VMEM((N_COARSE + 8, 16), jnp.float32),
            pltpu.VMEM((N_COARSE + 8, 16), jnp.int32),
            pltpu.VMEM((N_ALL, CHUNK), jnp.float32),
        ],
    )(_sc_sampler)
    zsT = sck(densT, zT, deltaT, uT, zeT)

    ptsT = pl.pallas_call(
        _pts_body,
        grid=(B // PTS_BS,),
        in_specs=[
            pl.BlockSpec((N_ALL, PTS_BS), lambda b: (0, b)),
            pl.BlockSpec((3, PTS_BS), lambda b: (0, b)),
            pl.BlockSpec((3, PTS_BS), lambda b: (0, b)),
        ],
        out_specs=pl.BlockSpec((3, N_ALL, PTS_BS), lambda b: (0, 0, b)),
        out_shape=jax.ShapeDtypeStruct((3, N_ALL, B), jnp.float32),
    )(zsT, origins.T, directions.T)

    return (ptsT.transpose(2, 1, 0), zsT.T[..., None])


# speedup vs baseline: 1.1986x; 1.1986x over previous
import dataclasses
import functools

import jax
import jax.numpy as jnp
from jax import lax
from jax.experimental import pallas as pl
from jax.experimental.pallas import tpu as pltpu
from jax.experimental.pallas import tpu_sc as plsc

N_COARSE = 64
N_FINE = 128
N_ALL = N_COARSE + N_FINE
MIN_DEPTH = 2.0
MAX_DEPTH = 6.0
NW = 32
CHUNK = 128
GROUPS = CHUNK // 16
PTS_BS = 1024


def _sc_sampler(densT, zT, deltaT, uT, zeT, zsT,
                dens_s, z_s, delta_s, u_s, ze_s, cdf_s, hist_s, outz_s):
    B = densT.shape[1]
    rays_per_w = B // NW
    nchunk = rays_per_w // CHUNK
    w_id = lax.axis_index("c") * 16 + lax.axis_index("s")
    lane = lax.iota(jnp.int32, 16)
    izeros = jnp.zeros((16,), jnp.int32)

    @pl.loop(0, N_COARSE)
    def _hz(j):
        hist_s[j, :] = izeros

    @pl.loop(0, nchunk)
    def _chunk(ci):
        col = w_id * rays_per_w + ci * CHUNK
        pltpu.sync_copy(densT.at[:, pl.ds(col, CHUNK)], dens_s)
        pltpu.sync_copy(zT.at[:, pl.ds(col, CHUNK)], z_s)
        pltpu.sync_copy(deltaT.at[:, pl.ds(col, CHUNK)], delta_s)
        pltpu.sync_copy(uT.at[:, pl.ds(col, CHUNK)], u_s)
        pltpu.sync_copy(zeT.at[:, pl.ds(col, CHUNK)],
                        ze_s.at[pl.ds(0, N_COARSE + 1)])

        for g in range(GROUPS):
            c0 = g * 16
            glane = lane + c0

            cdf_s[0, :] = jnp.zeros((16,), jnp.float32)

            def wstep(j, carry):
                T, wsum = carry
                x = dens_s[j, pl.ds(c0, 16)] * delta_s[j, pl.ds(c0, 16)]
                e = jnp.exp(-x)
                wj = (1.0 - e) * T + 1e-05
                wsum = wsum + wj
                cdf_s[j + 1, :] = wsum
                return T * (e + 1e-10), wsum

            ones = jnp.full((16,), 1.0, jnp.float32)
            _, wsum = lax.fori_loop(
                0, N_COARSE, wstep, (ones, jnp.zeros((16,), jnp.float32)))
            inv = ones / wsum
            thresh = 1e-05 * wsum

            h_grid = (MAX_DEPTH - MIN_DEPTH) / (N_COARSE - 1)
            s0 = MIN_DEPTH + 0.5 * h_grid
            inv_h = 1.0 / h_grid
            ione = jnp.full((16,), 1, jnp.int32)

            @pl.loop(0, N_FINE, step=2)
            def _q(q0):
                for q in (q0, q0 + 1):
                    uu = u_s[q, pl.ds(c0, 16)]
                    uw = uu * wsum
                    k = jnp.zeros((16,), jnp.int32)
                    for b in (64, 32, 16, 8, 4, 2, 1):
                        cand = jnp.minimum(k + b, N_COARSE)
                        val = plsc.load_gather(cdf_s, [cand, lane])
                        k = jnp.where(val <= uw, cand, k)
                    k1 = jnp.minimum(k + 1, N_COARSE)
                    cdf0 = plsc.load_gather(cdf_s, [k, lane])
                    cdf1 = plsc.load_gather(cdf_s, [k1, lane])
                    e0 = plsc.load_gather(ze_s, [k, glane])
                    e1 = plsc.load_gather(ze_s, [k1, glane])
                    dn = cdf1 - cdf0
                    num = uw - cdf0
                    t = jnp.where(dn < thresh, num * inv, num / dn)
                    fz = e0 + t * (e1 - e0)
                    m = ((fz - s0) * inv_h).astype(jnp.int32) + 1
                    m = jnp.clip(m, 1, N_COARSE - 2)
                    za = plsc.load_gather(z_s, [m - 1, glane])
                    zb = plsc.load_gather(z_s, [m, glane])
                    zc = plsc.load_gather(z_s, [m + 1, glane])
                    cnt = (m - 1) + (za <= fz).astype(jnp.int32) \
                        + (zb <= fz).astype(jnp.int32) \
                        + (zc <= fz).astype(jnp.int32)
                    plsc.addupdate_scatter(hist_s, [cnt, lane], ione)
                    plsc.store_scatter(outz_s, [cnt + q, glane], fz)

            def istep(i, H):
                H = H + hist_s[i, :]
                hist_s[i, :] = izeros
                zi = z_s[i, pl.ds(c0, 16)]
                plsc.store_scatter(outz_s, [H + i, glane], zi)
                return H

            lax.fori_loop(0, N_COARSE, istep, jnp.zeros((16,), jnp.int32))

        pltpu.sync_copy(outz_s, zsT.at[:, pl.ds(col, CHUNK)])


def _pts_body(zsT_ref, oT_ref, dT_ref, pts_ref):
    zs = zsT_ref[...]
    o = oT_ref[...]
    d = dT_ref[...]
    pts_ref[...] = o[:, None, :] + d[:, None, :] * zs[None, :, :]


TC_BS = 256
N_PAD = 256
BIG = 3.0e38


def _scan_last(x, op, identity):
    n = x.shape[-1]
    s = 1
    while s < n:
        pad = jnp.full(x.shape[:-1] + (s,), identity, x.dtype)
        x = op(x, jnp.concatenate([pad, x[..., :-s]], axis=-1))
        s *= 2
    return x


def _tc_body(dens_ref, z_ref, u_ref, zea_ref, zeb_ref, o_ref, d_ref,
             zs_ref, pts_ref):
    dens = dens_ref[...]
    z = z_ref[...]
    u = u_ref[...]

    delta = jnp.concatenate(
        [z[:, 1:] - z[:, :-1], jnp.full((TC_BS, 1), 1e10, jnp.float32)],
        axis=-1)
    alpha = 1.0 - jnp.exp(-dens * delta)
    am = 1.0 - alpha + 1e-10
    cp = _scan_last(am, jnp.multiply, jnp.float32(1.0))
    T = jnp.concatenate(
        [jnp.ones((TC_BS, 1), jnp.float32), cp[:, :-1]], axis=-1)
    w = alpha * T + 1e-05
    pdf = w / jnp.sum(w, axis=-1, keepdims=True)
    csum = _scan_last(pdf, jnp.add, jnp.float32(0.0))
    zea = zea_ref[...]
    zeb = zeb_ref[...]

    cdf0 = jnp.zeros((TC_BS, N_FINE), jnp.float32)
    e0 = jnp.broadcast_to(zea[:, :1], (TC_BS, N_FINE))
    cdf1 = jnp.broadcast_to(csum[:, N_COARSE - 1:], (TC_BS, N_FINE))
    e1 = jnp.broadcast_to(zeb[:, N_COARSE - 1:], (TC_BS, N_FINE))
    for j in range(N_COARSE):
        cs_j = csum[:, j:j + 1]
        ze_j = zeb[:, j:j + 1]
        le = u >= cs_j
        cdf0 = jnp.where(le, cs_j, cdf0)
        e0 = jnp.where(le, ze_j, e0)
        j2 = N_COARSE - 1 - j
        cs_j2 = csum[:, j2:j2 + 1]
        ze_j2 = zeb[:, j2:j2 + 1]
        gt = cs_j2 > u
        cdf1 = jnp.where(gt, cs_j2, cdf1)
        e1 = jnp.where(gt, ze_j2, e1)

    denom = cdf1 - cdf0
    denom = jnp.where(denom < 1e-05, jnp.float32(1.0), denom)
    t = (u - cdf0) / denom
    fine = e0 + t * (e1 - e0)

    s = jnp.concatenate(
        [z, jnp.full((TC_BS, N_COARSE), BIG, jnp.float32), fine], axis=-1)
    lane = jax.lax.broadcasted_iota(jnp.int32, (TC_BS, N_PAD), 1)
    step_sz = N_PAD // 2
    while step_sz >= 1:
        up = jnp.concatenate([s[:, step_sz:], s[:, :step_sz]], axis=-1)
        dn = jnp.concatenate([s[:, -step_sz:], s[:, :-step_sz]], axis=-1)
        is_lo = (lane & step_sz) == 0
        s = jnp.where(is_lo, jnp.minimum(s, up), jnp.maximum(s, dn))
        step_sz //= 2
    zs = s[:, :N_ALL]
    zs_ref[...] = zs
    o = o_ref[...]
    d = d_ref[...]
    pts_ref[...] = o[:, :, None] + d[:, :, None] * zs[:, None, :]


def _constants(B):
    rkey = jax.random.key(12345)
    rk1, rk2 = jax.random.split(rkey)
    zlin = jnp.linspace(MIN_DEPTH, MAX_DEPTH, N_COARSE, dtype=jnp.float32)
    z_vals = jnp.broadcast_to(zlin, (B, N_COARSE))
    mids = 0.5 * (z_vals[..., 1:] + z_vals[..., :-1])
    upper = jnp.concatenate([mids, z_vals[..., -1:]], axis=-1)
    lower = jnp.concatenate([z_vals[..., :1], mids], axis=-1)
    t_rand = jax.random.uniform(rk1, z_vals.shape, dtype=jnp.float32)
    z_vals = lower + (upper - lower) * t_rand
    u = jax.random.uniform(rk2, (B, N_FINE), dtype=jnp.float32)
    u_asc = jnp.sort(u, axis=-1)
    delta = jnp.concatenate(
        [z_vals[..., 1:] - z_vals[..., :-1],
         jnp.full((B, 1), 1e10, jnp.float32)], axis=-1)
    ze_mid = 0.5 * (z_vals[..., 1:] + z_vals[..., :-1])
    z_edges = jnp.concatenate(
        [
            z_vals[..., :1] - 0.5 * (z_vals[..., 1:2] - z_vals[..., :1]),
            ze_mid,
            z_vals[..., -1:] + 0.5 * (z_vals[..., -1:] - z_vals[..., -2:-1]),
        ],
        axis=-1,
    )
    return (z_vals.T, delta.T, u_asc.T, z_edges.T,
            z_vals, u_asc[:, ::-1], z_edges[:, :N_COARSE], z_edges[:, 1:])


B_SC = 12288


def kernel(origins, directions, coarse_density):
    B = origins.shape[0]
    with jax.ensure_compile_time_eval():
        (zT, deltaT, uT, zeT,
         z_rows, u_desc, zea_rows, zeb_rows) = _constants(B)
        zT, deltaT, uT, zeT = (x[:, :B_SC] for x in (zT, deltaT, uT, zeT))
        z_tc = z_rows[B_SC:]
        u_tc = u_desc[B_SC:]
        zea_tc = zea_rows[B_SC:]
        zeb_tc = zeb_rows[B_SC:]
    B_TC = B - B_SC

    dens = coarse_density[..., 0]
    densT = dens[:B_SC].T

    cp = pltpu.CompilerParams()
    if "needs_layout_passes" in pltpu.CompilerParams.__dataclass_fields__:
        cp = dataclasses.replace(cp, needs_layout_passes=False)
    sck = functools.partial(
        pl.kernel,
        out_type=jax.ShapeDtypeStruct((N_ALL, B_SC), jnp.float32),
        compiler_params=cp,
        mesh=plsc.VectorSubcoreMesh(core_axis_name="c", subcore_axis_name="s"),
        scratch_types=[
            pltpu.VMEM((N_COARSE, CHUNK), jnp.float32),
            pltpu.VMEM((N_COARSE, CHUNK), jnp.float32),
            pltpu.VMEM((N_COARSE, CHUNK), jnp.float32),
            pltpu.VMEM((N_FINE, CHUNK), jnp.float32),
            pltpu.VMEM((N_COARSE + 8, CHUNK), jnp.float32),
            pltpu.VMEM((N_COARSE + 8, 16), jnp.float32),
            pltpu.VMEM((N_COARSE + 8, 16), jnp.int32),
            pltpu.VMEM((N_ALL, CHUNK), jnp.float32),
        ],
    )(_sc_sampler)
    zsT = sck(densT, zT, deltaT, uT, zeT)

    zs_tc, pts_tc = pl.pallas_call(
        _tc_body,
        grid=(B_TC // TC_BS,),
        in_specs=[
            pl.BlockSpec((TC_BS, N_COARSE), lambda b: (b, 0)),
            pl.BlockSpec((TC_BS, N_COARSE), lambda b: (b, 0)),
            pl.BlockSpec((TC_BS, N_FINE), lambda b: (b, 0)),
            pl.BlockSpec((TC_BS, N_COARSE), lambda b: (b, 0)),
            pl.BlockSpec((TC_BS, N_COARSE), lambda b: (b, 0)),
            pl.BlockSpec((TC_BS, 3), lambda b: (b, 0)),
            pl.BlockSpec((TC_BS, 3), lambda b: (b, 0)),
        ],
        out_specs=[
            pl.BlockSpec((TC_BS, N_ALL), lambda b: (b, 0)),
            pl.BlockSpec((TC_BS, 3, N_ALL), lambda b: (b, 0, 0)),
        ],
        out_shape=[
            jax.ShapeDtypeStruct((B_TC, N_ALL), jnp.float32),
            jax.ShapeDtypeStruct((B_TC, 3, N_ALL), jnp.float32),
        ],
    )(dens[B_SC:], z_tc, u_tc, zea_tc, zeb_tc,
      origins[B_SC:], directions[B_SC:])

    ptsT_sc = pl.pallas_call(
        _pts_body,
        grid=(B_SC // PTS_BS,),
        in_specs=[
            pl.BlockSpec((N_ALL, PTS_BS), lambda b: (0, b)),
            pl.BlockSpec((3, PTS_BS), lambda b: (0, b)),
            pl.BlockSpec((3, PTS_BS), lambda b: (0, b)),
        ],
        out_specs=pl.BlockSpec((3, N_ALL, PTS_BS), lambda b: (0, 0, b)),
        out_shape=jax.ShapeDtypeStruct((3, N_ALL, B_SC), jnp.float32),
    )(zsT, origins[:B_SC].T, directions[:B_SC].T)

    all_pts = jnp.concatenate(
        [ptsT_sc.transpose(2, 1, 0), pts_tc.transpose(0, 2, 1)], axis=0)
    all_z = jnp.concatenate([zsT.T, zs_tc], axis=0)
    return (all_pts, all_z[..., None])
